# baseline (speedup 1.0000x reference)
import functools

import jax
import jax.numpy as jnp
from jax import lax
from jax.experimental import pallas as pl
from jax.experimental.pallas import tpu as pltpu
from jax.experimental.pallas import tpu_sc as plsc

N = 10000
N_PAD = 10240
E = 160000
E_PAD = 163840
D = 256
DH = 128
NC = 2
NS = 16
RPT = N_PAD // NS
K = 128
BPT_C = E_PAD // NS // K
DUMP = N_PAD - 1
NBUF = 4

_mesh = plsc.VectorSubcoreMesh(
    core_axis_name="c", subcore_axis_name="s", num_cores=NC, num_subcores=NS)


_DEG_KERNEL_CFG = dict(
    out_type=jax.ShapeDtypeStruct((NC, N_PAD, DH), jnp.float32),
    scratch_types=[
        pltpu.VMEM_SHARED((N_PAD, DH), jnp.float32),
        pltpu.VMEM((BPT_C, K), jnp.int32),
        pltpu.VMEM((K, DH), jnp.float32),
        pltpu.SemaphoreType.DMA,
        pltpu.SemaphoreType.DMA,
        pltpu.SemaphoreType.DMA,
        pltpu.SemaphoreType.DMA,
    ],
)
def _sc_degrees_body(idx2_hbm, ones_hbm, zeros_hbm, hist_hbm,
                     sp_h, idxv, ones_v, m0, m1, m2, m3):
    c = lax.axis_index("c")
    s = lax.axis_index("s")
    r0 = s * RPT
    sems = [m0, m1, m2, m3]

    pltpu.sync_copy(ones_hbm, ones_v)
    pltpu.sync_copy(idx2_hbm.at[c * NS + s], idxv)
    pltpu.sync_copy(zeros_hbm.at[pl.ds(r0, RPT)], sp_h.at[pl.ds(r0, RPT)])
    plsc.subcore_barrier()

    def round_body(g, carry):
        for j in range(NBUF):
            i = g * NBUF + j

            @pl.when(g > 0)
            def _():
                pltpu.make_async_copy(
                    ones_v, sp_h.at[idxv.at[0]], sems[j]).wait()

            pltpu.async_copy(ones_v, sp_h.at[idxv.at[i]], sems[j], add=True)
        return carry

    lax.fori_loop(0, BPT_C // NBUF, round_body, 0)
    for j in range(NBUF):
        pltpu.make_async_copy(ones_v, sp_h.at[idxv.at[0]], sems[j]).wait()
    plsc.subcore_barrier()

    pltpu.sync_copy(sp_h.at[pl.ds(r0, RPT)], hist_hbm.at[c, pl.ds(r0, RPT)])


_sc_degrees = pl.kernel(_sc_degrees_body, mesh=_mesh, **_DEG_KERNEL_CFG)


def _scale_body(feat_ref, hist_ref, fs_ref, nr_ref):
    h = hist_ref[...]
    deg_o = h[0, :, 0]
    deg_i = h[1, :, 0]
    norm_l = lax.rsqrt(jnp.maximum(deg_o, 1.0))
    norm_r = lax.rsqrt(jnp.maximum(deg_i, 1.0))
    f = feat_ref[...] * norm_l[:, None]
    fs_ref[0] = f[:, :DH]
    fs_ref[1] = f[:, DH:]
    nr_ref[...] = norm_r[:, None]


def _tc_scale(feat, hist):
    R = 1000
    grid = N // R
    return pl.pallas_call(
        _scale_body,
        grid=(grid,),
        in_specs=[
            pl.BlockSpec((R, D), lambda i: (i, 0)),
            pl.BlockSpec((NC, R, DH), lambda i: (0, i, 0)),
        ],
        out_specs=[
            pl.BlockSpec((NC, R, DH), lambda i: (0, i, 0)),
            pl.BlockSpec((R, 1), lambda i: (i, 0)),
        ],
        out_shape=[
            jax.ShapeDtypeStruct((NC, N, DH), jnp.float32),
            jax.ShapeDtypeStruct((N, 1), jnp.float32),
        ],
    )(feat, hist)


_AGG_KERNEL_CFG = dict(
    out_type=jax.ShapeDtypeStruct((NC, N_PAD, DH), jnp.float32),
    scratch_types=[
        pltpu.VMEM_SHARED((N_PAD, DH), jnp.float32),
        pltpu.VMEM((BPT_C, K), jnp.int32),
        pltpu.VMEM((K,), jnp.int32),
        pltpu.VMEM((K,), jnp.int32),
        pltpu.VMEM((K,), jnp.int32),
        pltpu.VMEM((K,), jnp.int32),
        pltpu.VMEM((K, DH), jnp.float32),
        pltpu.VMEM((K, DH), jnp.float32),
        pltpu.SemaphoreType.DMA,
        pltpu.SemaphoreType.DMA,
        pltpu.SemaphoreType.DMA,
        pltpu.SemaphoreType.DMA,
        pltpu.SemaphoreType.DMA,
        pltpu.SemaphoreType.DMA,
        pltpu.SemaphoreType.DMA,
        pltpu.SemaphoreType.DMA,
        pltpu.SemaphoreType.DMA,
        pltpu.SemaphoreType.DMA,
    ],
)
def _sc_aggregate_body(feat2_hbm, src2_hbm, dst3_hbm, zeros_hbm,
                       agg_hbm,
                       sp_agg, idxd, i0, i1, i2, i3, b0, b1,
                       n0, n1, n2, n3, g0, g1, h0, h1, t0, t1):
    c = lax.axis_index("c")
    s = lax.axis_index("s")
    r0 = s * RPT
    w = c * NS + s
    idxs = [i0, i1, i2, i3]
    isem = [n0, n1, n2, n3]
    bufs = [b0, b1]
    gsem = [g0, g1]
    gsem2 = [h0, h1]
    ssem = [t0, t1]

    def idx_load_start(i, sl):
        pltpu.async_copy(src2_hbm.at[w, i], idxs[sl], isem[sl])

    def idx_wait(sl):
        pltpu.make_async_copy(src2_hbm.at[w, 0], idxs[sl], isem[sl]).wait()

    KH = K // 2

    def gather_start(sl, bl):
        pltpu.async_copy(feat2_hbm.at[idxs[sl].at[pl.ds(0, KH)]],
                         bufs[bl].at[pl.ds(0, KH)], gsem[bl])
        pltpu.async_copy(feat2_hbm.at[idxs[sl].at[pl.ds(KH, KH)]],
                         bufs[bl].at[pl.ds(KH, KH)], gsem2[bl])

    def gather_wait(bl):
        pltpu.make_async_copy(
            feat2_hbm.at[idxs[0].at[pl.ds(0, KH)]],
            bufs[bl].at[pl.ds(0, KH)], gsem[bl]).wait()
        pltpu.make_async_copy(
            feat2_hbm.at[idxs[0].at[pl.ds(0, KH)]],
            bufs[bl].at[pl.ds(KH, KH)], gsem2[bl]).wait()

    def scat_start(i, bl):
        pltpu.async_copy(bufs[bl], sp_agg.at[idxd.at[i]], ssem[bl], add=True)

    def scat_wait(bl):
        pltpu.make_async_copy(
            bufs[bl], sp_agg.at[idxd.at[0]], ssem[bl]).wait()

    pltpu.sync_copy(dst3_hbm.at[s], idxd)
    idx_load_start(0, 0)
    idx_load_start(1, 1)
    pltpu.sync_copy(zeros_hbm.at[pl.ds(r0, RPT)], sp_agg.at[pl.ds(r0, RPT)])
    plsc.subcore_barrier()

    def round_body(g, carry):
        for j in range(4):
            i = g * 4 + j
            bl = j % 2
            bl1 = (j + 1) % 2

            if j >= 2:
                scat_wait(bl)
            else:
                @pl.when(g > 0)
                def _():
                    scat_wait(bl)

            if j >= 2:
                @pl.when(g < BPT_C // 4 - 1)
                def _():
                    idx_load_start(i + 2, (j + 2) % 4)
            else:
                idx_load_start(i + 2, (j + 2) % 4)

            idx_wait(j)
            gather_start(j, bl)

            if j >= 1:
                gather_wait(bl1)
                scat_start(i - 1, bl1)
            else:
                @pl.when(g > 0)
                def _():
                    gather_wait(bl1)
                    scat_start(i - 1, bl1)
        return carry

    lax.fori_loop(0, BPT_C // 4, round_body, 0)
    gather_wait((BPT_C - 1) % 2)
    scat_start(BPT_C - 1, (BPT_C - 1) % 2)
    scat_wait(0)
    scat_wait(1)
    plsc.subcore_barrier()

    pltpu.sync_copy(sp_agg.at[pl.ds(r0, RPT)], agg_hbm.at[c, pl.ds(r0, RPT)])


_sc_aggregate = pl.kernel(_sc_aggregate_body, mesh=_mesh, **_AGG_KERNEL_CFG)


def _matmul_body(agg_ref, w_ref, nr_ref, b_ref, out_ref):
    a = agg_ref[...]
    wm = w_ref[...]
    acc = jnp.dot(a[0], wm[:DH, :], preferred_element_type=jnp.float32)
    acc += jnp.dot(a[1], wm[DH:, :], preferred_element_type=jnp.float32)
    out_ref[...] = acc * nr_ref[...] + b_ref[...]


def _tc_matmul(agg2, W, nr, b2):
    R = 1000
    grid = N // R
    return pl.pallas_call(
        _matmul_body,
        grid=(grid,),
        in_specs=[
            pl.BlockSpec((NC, R, DH), lambda i: (0, i, 0)),
            pl.BlockSpec((D, D), lambda i: (0, 0)),
            pl.BlockSpec((R, 1), lambda i: (i, 0)),
            pl.BlockSpec((1, D), lambda i: (0, 0)),
        ],
        out_specs=pl.BlockSpec((R, D), lambda i: (i, 0)),
        out_shape=jax.ShapeDtypeStruct((N, D), jnp.float32),
    )(agg2, W, nr, b2)


@jax.jit
def kernel(feat, edge_index, W, b):
    src = edge_index[0].astype(jnp.int32)
    dst = edge_index[1].astype(jnp.int32)

    npad = E_PAD - E
    pad_dump = jnp.full((npad,), DUMP, jnp.int32)
    src_h = jnp.concatenate([src, pad_dump])
    dst_p = jnp.concatenate([dst, pad_dump])
    idx2 = jnp.concatenate([src_h, dst_p]).reshape(NC * NS, BPT_C, K)
    src_p0 = jnp.concatenate([src, jnp.zeros((npad,), jnp.int32)])
    src2 = jnp.concatenate([src_p0, src_p0 + N]).reshape(NC * NS, BPT_C, K)
    dst3 = dst_p.reshape(NS, BPT_C, K)

    ones = jnp.ones((K, DH), jnp.float32)
    zeros = jnp.zeros((N_PAD, DH), jnp.float32)

    hist = _sc_degrees(idx2, ones, zeros)
    fs2, nr = _tc_scale(feat, hist)

    feat_flat = fs2.reshape(NC * N, DH)
    agg2 = _sc_aggregate(feat_flat, src2, dst3, zeros)

    return _tc_matmul(agg2, W, nr, b.reshape(1, D))


# speedup vs baseline: 1.0038x; 1.0038x over previous
import jax
import jax.numpy as jnp
from jax import lax
from jax.experimental import pallas as pl
from jax.experimental.pallas import tpu as pltpu
from jax.experimental.pallas import tpu_sc as plsc

N = 10000
N_PAD = 10240
E = 160000
E_PAD = 163840
D = 256
DH = 128
NC = 2
NS = 16
RPT = N_PAD // NS
K = 128
BPT_C = E_PAD // NS // K
DUMP = N_PAD - 1
NBUF = 4

_mesh = plsc.VectorSubcoreMesh(
    core_axis_name="c", subcore_axis_name="s", num_cores=NC, num_subcores=NS)


_DEG_KERNEL_CFG = dict(
    out_type=jax.ShapeDtypeStruct((NC, N_PAD, DH), jnp.float32),
    scratch_types=[
        pltpu.VMEM_SHARED((N_PAD, DH), jnp.float32),
        pltpu.VMEM((BPT_C, K), jnp.int32),
        pltpu.VMEM((K, DH), jnp.float32),
        pltpu.SemaphoreType.DMA,
        pltpu.SemaphoreType.DMA,
        pltpu.SemaphoreType.DMA,
        pltpu.SemaphoreType.DMA,
    ],
)
def _sc_degrees_body(idx2_hbm, ones_hbm, zeros_hbm, hist_hbm,
                     sp_h, idxv, ones_v, m0, m1, m2, m3):
    c = lax.axis_index("c")
    s = lax.axis_index("s")
    r0 = s * RPT
    sems = [m0, m1, m2, m3]

    pltpu.sync_copy(ones_hbm, ones_v)
    pltpu.sync_copy(idx2_hbm.at[c * NS + s], idxv)
    pltpu.sync_copy(zeros_hbm.at[pl.ds(r0, RPT)], sp_h.at[pl.ds(r0, RPT)])
    plsc.subcore_barrier()

    def round_body(g, carry):
        for j in range(NBUF):
            i = g * NBUF + j

            @pl.when(g > 0)
            def _():
                pltpu.make_async_copy(
                    ones_v, sp_h.at[idxv.at[0]], sems[j]).wait()

            pltpu.async_copy(ones_v, sp_h.at[idxv.at[i]], sems[j], add=True)
        return carry

    lax.fori_loop(0, BPT_C // NBUF, round_body, 0)
    for j in range(NBUF):
        pltpu.make_async_copy(ones_v, sp_h.at[idxv.at[0]], sems[j]).wait()
    plsc.subcore_barrier()

    pltpu.sync_copy(sp_h.at[pl.ds(r0, RPT)], hist_hbm.at[c, pl.ds(r0, RPT)])


_sc_degrees = pl.kernel(_sc_degrees_body, mesh=_mesh, **_DEG_KERNEL_CFG)


def _scale_body(feat_ref, hist_ref, fs_ref, nr_ref):
    h = hist_ref[...]
    deg_o = h[0, :, 0]
    deg_i = h[1, :, 0]
    norm_l = lax.rsqrt(jnp.maximum(deg_o, 1.0))
    norm_r = lax.rsqrt(jnp.maximum(deg_i, 1.0))
    f = feat_ref[...] * norm_l[:, None]
    fs_ref[0] = f[:, :DH]
    fs_ref[1] = f[:, DH:]
    nr_ref[...] = norm_r[:, None]


def _tc_scale(feat, hist):
    R = 1000
    grid = N // R
    return pl.pallas_call(
        _scale_body,
        grid=(grid,),
        in_specs=[
            pl.BlockSpec((R, D), lambda i: (i, 0)),
            pl.BlockSpec((NC, R, DH), lambda i: (0, i, 0)),
        ],
        out_specs=[
            pl.BlockSpec((NC, R, DH), lambda i: (0, i, 0)),
            pl.BlockSpec((R, 1), lambda i: (i, 0)),
        ],
        out_shape=[
            jax.ShapeDtypeStruct((NC, N, DH), jnp.float32),
            jax.ShapeDtypeStruct((N, 1), jnp.float32),
        ],
    )(feat, hist)


_AGG_KERNEL_CFG = dict(
    out_type=jax.ShapeDtypeStruct((NC, N_PAD, DH), jnp.float32),
    scratch_types=[
        pltpu.VMEM_SHARED((N_PAD, DH), jnp.float32),
        pltpu.VMEM((BPT_C, K), jnp.int32),
        pltpu.VMEM((K,), jnp.int32),
        pltpu.VMEM((K,), jnp.int32),
        pltpu.VMEM((K,), jnp.int32),
        pltpu.VMEM((K,), jnp.int32),
        pltpu.VMEM((K, DH), jnp.float32),
        pltpu.VMEM((K, DH), jnp.float32),
        pltpu.SemaphoreType.DMA,
        pltpu.SemaphoreType.DMA,
        pltpu.SemaphoreType.DMA,
        pltpu.SemaphoreType.DMA,
        pltpu.SemaphoreType.DMA,
        pltpu.SemaphoreType.DMA,
        pltpu.SemaphoreType.DMA,
        pltpu.SemaphoreType.DMA,
    ],
)
def _sc_aggregate_body(feat2_hbm, src2_hbm, dst3_hbm, zeros_hbm,
                       agg_hbm,
                       sp_agg, idxd, i0, i1, i2, i3, b0, b1,
                       n0, n1, n2, n3, g0, g1, t0, t1):
    c = lax.axis_index("c")
    s = lax.axis_index("s")
    r0 = s * RPT
    w = c * NS + s
    idxs = [i0, i1, i2, i3]
    isem = [n0, n1, n2, n3]
    bufs = [b0, b1]
    gsem = [g0, g1]
    ssem = [t0, t1]

    def idx_load_start(i, sl):
        pltpu.async_copy(src2_hbm.at[w, i], idxs[sl], isem[sl])

    def idx_wait(sl):
        pltpu.make_async_copy(src2_hbm.at[w, 0], idxs[sl], isem[sl]).wait()

    def gather_start(sl, bl):
        pltpu.async_copy(feat2_hbm.at[idxs[sl]], bufs[bl], gsem[bl])

    def gather_wait(bl):
        pltpu.make_async_copy(
            feat2_hbm.at[idxs[0]], bufs[bl], gsem[bl]).wait()

    def scat_start(i, bl):
        pltpu.async_copy(bufs[bl], sp_agg.at[idxd.at[i]], ssem[bl], add=True)

    def scat_wait(bl):
        pltpu.make_async_copy(
            bufs[bl], sp_agg.at[idxd.at[0]], ssem[bl]).wait()

    pltpu.sync_copy(dst3_hbm.at[s], idxd)
    idx_load_start(0, 0)
    idx_load_start(1, 1)
    pltpu.sync_copy(zeros_hbm.at[pl.ds(r0, RPT)], sp_agg.at[pl.ds(r0, RPT)])
    plsc.subcore_barrier()

    def round_body(g, carry):
        for j in range(4):
            i = g * 4 + j
            bl = j % 2
            bl1 = (j + 1) % 2

            if j >= 2:
                scat_wait(bl)
            else:
                @pl.when(g > 0)
                def _():
                    scat_wait(bl)

            if j >= 2:
                @pl.when(g < BPT_C // 4 - 1)
                def _():
                    idx_load_start(i + 2, (j + 2) % 4)
            else:
                idx_load_start(i + 2, (j + 2) % 4)

            idx_wait(j)
            gather_start(j, bl)

            if j >= 1:
                gather_wait(bl1)
                scat_start(i - 1, bl1)
            else:
                @pl.when(g > 0)
                def _():
                    gather_wait(bl1)
                    scat_start(i - 1, bl1)
        return carry

    lax.fori_loop(0, BPT_C // 4, round_body, 0)
    gather_wait((BPT_C - 1) % 2)
    scat_start(BPT_C - 1, (BPT_C - 1) % 2)
    scat_wait(0)
    scat_wait(1)
    plsc.subcore_barrier()

    pltpu.sync_copy(sp_agg.at[pl.ds(r0, RPT)], agg_hbm.at[c, pl.ds(r0, RPT)])


_sc_aggregate = pl.kernel(_sc_aggregate_body, mesh=_mesh, **_AGG_KERNEL_CFG)


def _matmul_body(agg_ref, w_ref, nr_ref, b_ref, out_ref):
    a = agg_ref[...]
    wm = w_ref[...]
    acc = jnp.dot(a[0], wm[:DH, :], preferred_element_type=jnp.float32)
    acc += jnp.dot(a[1], wm[DH:, :], preferred_element_type=jnp.float32)
    out_ref[...] = acc * nr_ref[...] + b_ref[...]


def _tc_matmul(agg2, W, nr, b2):
    R = 1000
    grid = N // R
    return pl.pallas_call(
        _matmul_body,
        grid=(grid,),
        in_specs=[
            pl.BlockSpec((NC, R, DH), lambda i: (0, i, 0)),
            pl.BlockSpec((D, D), lambda i: (0, 0)),
            pl.BlockSpec((R, 1), lambda i: (i, 0)),
            pl.BlockSpec((1, D), lambda i: (0, 0)),
        ],
        out_specs=pl.BlockSpec((R, D), lambda i: (i, 0)),
        out_shape=jax.ShapeDtypeStruct((N, D), jnp.float32),
    )(agg2, W, nr, b2)


@jax.jit
def kernel(feat, edge_index, W, b):
    src = edge_index[0].astype(jnp.int32)
    dst = edge_index[1].astype(jnp.int32)

    npad = E_PAD - E
    pad_dump = jnp.full((npad,), DUMP, jnp.int32)
    src_h = jnp.concatenate([src, pad_dump])
    dst_p = jnp.concatenate([dst, pad_dump])
    idx2 = jnp.concatenate([src_h, dst_p]).reshape(NC * NS, BPT_C, K)
    src_p0 = jnp.concatenate([src, jnp.zeros((npad,), jnp.int32)])
    src2 = jnp.concatenate([src_p0, src_p0 + N]).reshape(NC * NS, BPT_C, K)
    dst3 = dst_p.reshape(NS, BPT_C, K)

    ones = jnp.ones((K, DH), jnp.float32)
    zeros = jnp.zeros((N_PAD, DH), jnp.float32)

    hist = _sc_degrees(idx2, ones, zeros)
    fs2, nr = _tc_scale(feat, hist)

    feat_flat = fs2.reshape(NC * N, DH)
    agg2 = _sc_aggregate(feat_flat, src2, dst3, zeros)

    return _tc_matmul(agg2, W, nr, b.reshape(1, D))
